# Initial kernel scaffold; baseline (speedup 1.0000x reference)
import jax
import jax.numpy as jnp
from jax.experimental import pallas as pl


def kernel(x, edge_index, edge_weight, W0l, b0l, W0r, b0r, W1l, b1l, W1r, b1r):
    raise NotImplementedError("write your pallas kernel here")


# speedup vs baseline: 6.5030x; 6.5030x over previous
import functools

import jax
import jax.numpy as jnp
from jax import lax
from jax.experimental import pallas as pl
from jax.experimental.pallas import tpu as pltpu
from jax.experimental.pallas import tpu_sc as plsc

N = 10000
E = 320000
D = 128
NC = 2
NS = 16
NW = NC * NS
EPW = E // NW
CH = 80
K = EPW // CH
SL = 624
TB = NS * SL
TAIL = N - TB

_f32 = jnp.float32
_i32 = jnp.int32


def _make_sc_agg(with_count):
    out_type = [jax.ShapeDtypeStruct((NC, N, D), _f32)]
    scratch = [
        pltpu.VMEM((EPW,), _i32),
        pltpu.VMEM((K, CH), _i32),
        pltpu.VMEM((EPW,), _f32),
        pltpu.VMEM((CH, D), _f32),
        pltpu.VMEM_SHARED((N, D), _f32),
        pltpu.SemaphoreType.DMA,
    ]
    if with_count:
        out_type.append(jax.ShapeDtypeStruct((NC * N,), _f32))
        scratch += [
            pltpu.VMEM((CH,), _f32),
            pltpu.VMEM((SL,), _f32),
            pltpu.VMEM_SHARED((N,), _f32),
        ]
    mesh = plsc.VectorSubcoreMesh(core_axis_name="c", subcore_axis_name="s")

    def body(x_hbm, src_hbm, dst_hbm, w_hbm, *rest):
        if with_count:
            (part_out, cnt_out, src_v, dst_v, w_v, rows_v, acc_sh, sem,
             ones_v, zc_v, cnt_sh) = rest
        else:
            part_out, src_v, dst_v, w_v, rows_v, acc_sh, sem = rest
        c = lax.axis_index("c")
        s = lax.axis_index("s")
        wid = s * NC + c
        base = s * SL
        is_last = s == NS - 1

        pltpu.sync_copy(src_hbm.at[wid], src_v)
        pltpu.sync_copy(dst_hbm.at[wid], dst_v)
        pltpu.sync_copy(w_hbm.at[wid], w_v)

        zeros16 = jnp.zeros((16,), _f32)

        def zrow(r, _):
            for g in range(D // 16):
                rows_v[r, pl.ds(g * 16, 16)] = zeros16
            return 0

        lax.fori_loop(0, CH, zrow, 0)
        n_full, rem = divmod(SL, CH)
        for i in range(n_full):
            pltpu.sync_copy(rows_v, acc_sh.at[pl.ds(base + i * CH, CH)])
        if rem:
            pltpu.sync_copy(rows_v.at[pl.ds(0, rem)],
                            acc_sh.at[pl.ds(base + n_full * CH, rem)])

        @pl.when(is_last)
        def _():
            pltpu.sync_copy(rows_v.at[pl.ds(0, TAIL)],
                            acc_sh.at[pl.ds(TB, TAIL)])

        if with_count:
            ones16 = jnp.ones((16,), _f32)
            for r in range(CH // 16):
                ones_v[pl.ds(r * 16, 16)] = ones16
            for r in range(SL // 16):
                zc_v[pl.ds(r * 16, 16)] = zeros16
            pltpu.sync_copy(zc_v, cnt_sh.at[pl.ds(base, SL)])

            @pl.when(is_last)
            def _():
                pltpu.sync_copy(zc_v.at[pl.ds(0, TAIL)],
                                cnt_sh.at[pl.ds(TB, TAIL)])

        plsc.subcore_barrier()

        def chunk(j, _):
            pltpu.async_copy(x_hbm.at[src_v.at[pl.ds(j * CH, CH)]],
                             rows_v, sem).wait()

            for r0 in range(0, CH, 16):
                wvec = w_v[pl.ds(j * CH + r0, 16)]
                for rr in range(16):
                    splat = jnp.broadcast_to(wvec[rr], (16,))
                    for g in range(D // 16):
                        sl = pl.ds(g * 16, 16)
                        rows_v[r0 + rr, sl] = rows_v[r0 + rr, sl] * splat

            pltpu.sync_copy(rows_v, acc_sh.at[dst_v.at[j]], add=True)
            if with_count:
                pltpu.sync_copy(ones_v, cnt_sh.at[dst_v.at[j]], add=True)
            return 0

        lax.fori_loop(0, K, chunk, 0)

        plsc.subcore_barrier()

        pltpu.sync_copy(acc_sh.at[pl.ds(base, SL)],
                        part_out.at[c, pl.ds(base, SL)])

        @pl.when(is_last)
        def _():
            pltpu.sync_copy(acc_sh.at[pl.ds(TB, TAIL)],
                            part_out.at[c, pl.ds(TB, TAIL)])

        if with_count:
            pltpu.sync_copy(cnt_sh.at[pl.ds(base, SL)], zc_v)
            pltpu.sync_copy(zc_v, cnt_out.at[pl.ds(c * N + base, SL)])

            @pl.when(is_last)
            def _():
                pltpu.sync_copy(cnt_sh.at[pl.ds(TB, TAIL)],
                                zc_v.at[pl.ds(0, TAIL)])
                pltpu.sync_copy(zc_v.at[pl.ds(0, TAIL)],
                                cnt_out.at[pl.ds(c * N + TB, TAIL)])

    return pl.kernel(body, out_type=tuple(out_type) if with_count else out_type[0],
                     mesh=mesh, scratch_types=scratch)


_sc_agg_cnt_k = _make_sc_agg(True)
_sc_agg_k = _make_sc_agg(False)


def _tc_dense(parts, cnt16, xin, Wl, Wr, bsum, relu):
    def body(p_ref, c_ref, x_ref, wl_ref, wr_ref, b_ref, o_ref):
        cnt = c_ref[0, :] + c_ref[1, :]
        inv = 1.0 / jnp.maximum(cnt, 1.0)
        agg = (p_ref[0] + p_ref[1]) * inv[:, None]
        y = (jnp.dot(agg, wl_ref[...], preferred_element_type=_f32)
             + jnp.dot(x_ref[...], wr_ref[...], preferred_element_type=_f32)
             + b_ref[...])
        o_ref[...] = jnp.maximum(y, 0.0) if relu else y

    return pl.pallas_call(
        body,
        out_shape=jax.ShapeDtypeStruct((N, D), _f32),
    )(parts, cnt16, xin, Wl, Wr, bsum)


@jax.jit
def kernel(x, edge_index, edge_weight, W0l, b0l, W0r, b0r, W1l, b1l, W1r, b1r):
    src = edge_index[0].astype(_i32).reshape(NW, EPW)
    dst = edge_index[1].astype(_i32).reshape(NW, K, CH)
    w2 = edge_weight.astype(_f32).reshape(NW, EPW)

    parts1, cntflat = _sc_agg_cnt_k(x, src, dst, w2)
    cnt2 = cntflat.reshape(NC, N)
    h = _tc_dense(parts1, cnt2, x, W0l, W0r,
                  (b0l + b0r).reshape(1, D), relu=True)
    parts2 = _sc_agg_k(h, src, dst, w2)
    out = _tc_dense(parts2, cnt2, h, W1l, W1r,
                    (b1l + b1r).reshape(1, D), relu=False)
    return out
